# baseline (speedup 1.0000x reference)
import functools

import jax
import jax.numpy as jnp
from jax import lax
from jax.experimental import pallas as pl
from jax.experimental.pallas import tpu as pltpu
from jax.experimental.pallas import tpu_sc as plsc

HIDDEN_DIM = 64
PADDED_DIM = 128
NUM_CORES = 2
NUM_SUBCORES = 16
NUM_WORKERS = NUM_CORES * NUM_SUBCORES
CHUNK = 128

PAD_BLOCK = 8000
EPI_BATCH = 128


def _pad_body(x_ref, o_ref):
    o_ref[:, :HIDDEN_DIM] = x_ref[...]
    o_ref[:, HIDDEN_DIM:] = jnp.zeros_like(o_ref[:, HIDDEN_DIM:])


def _epilogue_body(x_ref, o_ref, seq):
    x = x_ref[...]
    o_ref[...] = x[:, :HIDDEN_DIM].reshape(EPI_BATCH, seq, HIDDEN_DIM)


def kernel(input_ids, embed_tokens_weight):
    batch, seq = input_ids.shape
    vocab = embed_tokens_weight.shape[0]
    num_idx = batch * seq
    per_worker = num_idx // NUM_WORKERS
    flat_ids = input_ids.reshape(num_idx)

    table128 = pl.pallas_call(
        _pad_body,
        grid=(vocab // PAD_BLOCK,),
        in_specs=[
            pl.BlockSpec((PAD_BLOCK, HIDDEN_DIM), lambda i: (i, 0)),
        ],
        out_specs=pl.BlockSpec((PAD_BLOCK, PADDED_DIM), lambda i: (i, 0)),
        out_shape=jax.ShapeDtypeStruct((vocab, PADDED_DIM), jnp.float32),
    )(embed_tokens_weight)

    mesh = plsc.VectorSubcoreMesh(core_axis_name="c", subcore_axis_name="s")

    @functools.partial(
        pl.kernel,
        mesh=mesh,
        out_type=jax.ShapeDtypeStruct((num_idx, PADDED_DIM), jnp.float32),
        scratch_types=[
            pltpu.VMEM((CHUNK,), jnp.int32),
            pltpu.VMEM((CHUNK, PADDED_DIM), jnp.float32),
            pltpu.SemaphoreType.DMA,
        ],
    )
    def gather_kernel(table_hbm, idx_hbm, out_hbm, idx_v, rows_v, sem):
        wid = lax.axis_index("s") * NUM_CORES + lax.axis_index("c")
        base = wid * per_worker

        @pl.loop(0, per_worker, step=CHUNK)
        def _(c):
            pltpu.sync_copy(idx_hbm.at[pl.ds(base + c, CHUNK)], idx_v)
            pltpu.async_copy(table_hbm.at[idx_v], rows_v, sem).wait()
            pltpu.sync_copy(rows_v, out_hbm.at[pl.ds(base + c, CHUNK)])

    fat = gather_kernel(table128, flat_ids)

    out = pl.pallas_call(
        functools.partial(_epilogue_body, seq=seq),
        grid=(batch // EPI_BATCH,),
        in_specs=[
            pl.BlockSpec((EPI_BATCH * seq, PADDED_DIM), lambda i: (i, 0)),
        ],
        out_specs=pl.BlockSpec((EPI_BATCH, seq, HIDDEN_DIM), lambda i: (i, 0, 0)),
        out_shape=jax.ShapeDtypeStruct((batch, seq, HIDDEN_DIM), jnp.float32),
    )(fat)
    return out


# speedup vs baseline: 1.2751x; 1.2751x over previous
import functools

import jax
import jax.numpy as jnp
from jax import lax
from jax.experimental import pallas as pl
from jax.experimental.pallas import tpu as pltpu
from jax.experimental.pallas import tpu_sc as plsc

HIDDEN_DIM = 64
PADDED_DIM = 128
NUM_CORES = 2
NUM_SUBCORES = 16
NUM_WORKERS = NUM_CORES * NUM_SUBCORES
CHUNK = 128


def kernel(input_ids, embed_tokens_weight):
    batch, seq = input_ids.shape
    num_idx = batch * seq
    per_worker = num_idx // NUM_WORKERS
    n_iter = per_worker // CHUNK
    flat_ids = input_ids.reshape(num_idx)
    table128 = jnp.pad(embed_tokens_weight, ((0, 0), (0, PADDED_DIM - HIDDEN_DIM)))

    mesh = plsc.VectorSubcoreMesh(core_axis_name="c", subcore_axis_name="s")

    @functools.partial(
        pl.kernel,
        mesh=mesh,
        out_type=jax.ShapeDtypeStruct((num_idx, PADDED_DIM), jnp.float32),
        scratch_types=[
            pltpu.VMEM((2, CHUNK), jnp.int32),
            pltpu.VMEM((2, CHUNK, PADDED_DIM), jnp.float32),
            pltpu.SemaphoreType.DMA((2,)),
            pltpu.SemaphoreType.DMA((2,)),
            pltpu.SemaphoreType.DMA((2,)),
        ],
    )
    def gather_kernel(table_hbm, idx_hbm, out_hbm, idx_v, rows_v, sem_idx,
                      sem_gat, sem_out):
        wid = lax.axis_index("s") * NUM_CORES + lax.axis_index("c")
        base = wid * per_worker

        def idx_copy(it, buf):
            return pltpu.make_async_copy(
                idx_hbm.at[pl.ds(base + it * CHUNK, CHUNK)],
                idx_v.at[buf],
                sem_idx.at[buf],
            )

        def gat_copy(buf):
            return pltpu.make_async_copy(
                table_hbm.at[idx_v.at[buf]],
                rows_v.at[buf],
                sem_gat.at[buf],
            )

        def out_copy(it, buf):
            return pltpu.make_async_copy(
                rows_v.at[buf],
                out_hbm.at[pl.ds(base + it * CHUNK, CHUNK)],
                sem_out.at[buf],
            )

        idx_copy(0, 0).start()

        @pl.loop(0, n_iter, step=2)
        def _(i):
            for b in range(2):
                it = i + b
                idx_copy(it, b).wait()
                @pl.when(it >= 2)
                def _():
                    out_copy(it - 2, b).wait()

                gat_copy(b).start()

                @pl.when(it + 1 < n_iter)
                def _():
                    idx_copy(it + 1, 1 - b).start()

                @pl.when(it >= 1)
                def _():
                    gat_copy(1 - b).wait()
                    out_copy(it - 1, 1 - b).start()

        last = n_iter - 1
        gat_copy(last % 2).wait()
        out_copy(last, last % 2).start()
        out_copy(last - 1, (last - 1) % 2).wait()
        out_copy(last, last % 2).wait()

    fat = gather_kernel(table128, flat_ids)
    return fat[:, :HIDDEN_DIM].reshape(batch, seq, HIDDEN_DIM)
